# baseline (speedup 1.0000x reference)
import jax
import jax.numpy as jnp
from jax import lax
from jax.experimental import pallas as pl
from jax.experimental.pallas import tpu as pltpu
from jax.experimental.pallas import tpu_sc as plsc

_B, _L, _D = 4, 2048, 768
_NPOS = 30
_NW = 32
_ROWS = _B * _L
_RPW = _ROWS // _NW
_CHUNK = 64
_NCHUNK = _RPW // _CHUNK
_CT = _D // 128


def _sc_body(x_hbm, idx_hbm, tab_hbm, out_hbm,
             idx_v, tab_v, buf0, buf1, sin0, sin1, sout0, sout1, stab):
    wid = lax.axis_index("s") * 2 + lax.axis_index("c")
    base = wid * _RPW
    bufs = (buf0, buf1)
    sins = (sin0, sin1)
    souts = (sout0, sout1)
    iota = lax.iota(jnp.int32, 16)

    def start_load(c):
        b = c & 1
        r0 = base + c * _CHUNK
        return pltpu.async_copy(
            x_hbm.at[pl.ds(r0, _CHUNK), :], bufs[b], sins[b])

    def start_store(c):
        b = c & 1
        r0 = base + c * _CHUNK
        return pltpu.async_copy(
            bufs[b], out_hbm.at[pl.ds(r0, _CHUNK), :], souts[b])

    loads = {0: start_load(0)}
    tcopies = [
        pltpu.async_copy(tab_hbm.at[i, :], tab_v.at[pl.ds(i * _D, _D)], stab)
        for i in range(_NPOS)
    ]
    tcopies.append(
        pltpu.async_copy(idx_hbm.at[pl.ds(base, _RPW)], idx_v, stab))
    zv = jnp.zeros((16,), jnp.float32)
    for k in range(_NPOS * _D, (_NPOS + 2) * _D, 16):
        tab_v[pl.ds(k, 16)] = zv
    for t in tcopies:
        t.wait()
    stores = {}
    for c in range(_NCHUNK):
        b = c & 1
        loads.pop(c).wait()
        buf = bufs[b]

        def row_body(rp, carry, _c=c, _buf=buf):
            rr = [rp * 4 + k for k in range(4)]
            ts = [plsc.load_gather(
                idx_v, [jnp.broadcast_to(_c * _CHUNK + r, (16,))])
                for r in rr]
            pad = jnp.full((16,), _NPOS, jnp.int32)
            ts = [
                jnp.where(
                    jnp.broadcast_to(
                        ((base + _c * _CHUNK + r) & (_L - 1)) == 0, (16,)),
                    pad, t)
                for r, t in zip(rr, ts)
            ]
            aa = [t * _D + iota for t in ts]

            @plsc.parallel_loop(0, _D, 16, unroll=4)
            def jbody(j, _aa=aa, _rr=rr, _b=_buf):
                jv = jnp.broadcast_to(j, (16,)).astype(jnp.int32)
                for k in range(4):
                    v = plsc.load_gather(tab_v, [_aa[k] + jv])
                    plsc.addupdate(_b.at[_rr[k], pl.ds(j, 16)], v)

            return carry

        lax.fori_loop(0, _CHUNK // 4, row_body, 0)

        stores[c] = start_store(c)
        if c + 1 < _NCHUNK:
            if c - 1 >= 0:
                stores.pop(c - 1).wait()
            loads[c + 1] = start_load(c + 1)
    stores.pop(_NCHUNK - 1).wait()


def kernel(inputs, times, pos_table):
    x = inputs.reshape(_ROWS, _D)
    idx = times.astype(jnp.int32).reshape(_ROWS)
    tab = pos_table.astype(jnp.float32)

    mesh = plsc.VectorSubcoreMesh(core_axis_name="c", subcore_axis_name="s")
    f = pl.kernel(
        _sc_body,
        out_type=jax.ShapeDtypeStruct((_ROWS, _D), jnp.float32),
        mesh=mesh,
        compiler_params=pltpu.CompilerParams(
            use_tc_tiling_on_sc=True, needs_layout_passes=False
        ),
        scratch_types=[
            pltpu.VMEM((_RPW,), jnp.int32),
            pltpu.VMEM(((_NPOS + 2) * _D,), jnp.float32),
            pltpu.VMEM((_CHUNK, _D), jnp.float32),
            pltpu.VMEM((_CHUNK, _D), jnp.float32),
            pltpu.SemaphoreType.DMA,
            pltpu.SemaphoreType.DMA,
            pltpu.SemaphoreType.DMA,
            pltpu.SemaphoreType.DMA,
            pltpu.SemaphoreType.DMA,
        ],
    )
    out = f(x, idx, tab)
    return out.reshape(_B, _L, _D)


# speedup vs baseline: 1.0049x; 1.0049x over previous
import jax
import jax.numpy as jnp
from jax import lax
from jax.experimental import pallas as pl
from jax.experimental.pallas import tpu as pltpu
from jax.experimental.pallas import tpu_sc as plsc

_B, _L, _D = 4, 2048, 768
_NPOS = 30
_NW = 32
_ROWS = _B * _L
_RPW = _ROWS // _NW
_CHUNK = 64
_NCHUNK = _RPW // _CHUNK
_CT = _D // 128


def _sc_body(x_hbm, idx_hbm, tab_hbm, out_hbm,
             idx_v, tab_v, buf0, buf1, sin0, sin1, sout0, sout1, stab):
    wid = lax.axis_index("s") * 2 + lax.axis_index("c")
    base = wid * _RPW
    bufs = (buf0, buf1)
    sins = (sin0, sin1)
    souts = (sout0, sout1)
    iota = lax.iota(jnp.int32, 16)

    def start_load(c):
        b = c & 1
        r0 = base + c * _CHUNK
        return pltpu.async_copy(
            x_hbm.at[pl.ds(r0, _CHUNK), :], bufs[b], sins[b])

    def start_store(c):
        b = c & 1
        r0 = base + c * _CHUNK
        return pltpu.async_copy(
            bufs[b], out_hbm.at[pl.ds(r0, _CHUNK), :], souts[b])

    loads = {0: start_load(0)}
    tcopies = [
        pltpu.async_copy(tab_hbm.at[i, :], tab_v.at[pl.ds(i * _D, _D)], stab)
        for i in range(_NPOS)
    ]
    tcopies.append(
        pltpu.async_copy(
            idx_hbm.at[wid >> 3, pl.ds((wid & 7) * _RPW, _RPW)], idx_v, stab))
    zv = jnp.zeros((16,), jnp.float32)
    for k in range(_NPOS * _D, (_NPOS + 2) * _D, 16):
        tab_v[pl.ds(k, 16)] = zv
    for t in tcopies:
        t.wait()
    stores = {}
    for c in range(_NCHUNK):
        b = c & 1
        loads.pop(c).wait()
        buf = bufs[b]

        def row_body(rp, carry, _c=c, _buf=buf):
            rr = [rp * 4 + k for k in range(4)]
            ts = [plsc.load_gather(
                idx_v, [jnp.broadcast_to(_c * _CHUNK + r, (16,))])
                for r in rr]
            pad = jnp.full((16,), _NPOS, jnp.int32)
            ts = [
                jnp.where(
                    jnp.broadcast_to(
                        ((base + _c * _CHUNK + r) & (_L - 1)) == 0, (16,)),
                    pad, t)
                for r, t in zip(rr, ts)
            ]
            aa = [t * _D + iota for t in ts]

            @plsc.parallel_loop(0, _D, 16, unroll=4)
            def jbody(j, _aa=aa, _rr=rr, _b=_buf):
                jv = jnp.broadcast_to(j, (16,)).astype(jnp.int32)
                for k in range(4):
                    v = plsc.load_gather(tab_v, [_aa[k] + jv])
                    plsc.addupdate(_b.at[_rr[k], pl.ds(j, 16)], v)

            return carry

        lax.fori_loop(0, _CHUNK // 4, row_body, 0)

        stores[c] = start_store(c)
        if c + 1 < _NCHUNK:
            if c - 1 >= 0:
                stores.pop(c - 1).wait()
            loads[c + 1] = start_load(c + 1)
    stores.pop(_NCHUNK - 1).wait()


def kernel(inputs, times, pos_table):
    x = inputs.reshape(_ROWS, _D)
    idx = times.astype(jnp.int32)
    tab = pos_table.astype(jnp.float32)

    mesh = plsc.VectorSubcoreMesh(core_axis_name="c", subcore_axis_name="s")
    f = pl.kernel(
        _sc_body,
        out_type=jax.ShapeDtypeStruct((_ROWS, _D), jnp.float32),
        mesh=mesh,
        compiler_params=pltpu.CompilerParams(
            use_tc_tiling_on_sc=True, needs_layout_passes=False
        ),
        scratch_types=[
            pltpu.VMEM((_RPW,), jnp.int32),
            pltpu.VMEM(((_NPOS + 2) * _D,), jnp.float32),
            pltpu.VMEM((_CHUNK, _D), jnp.float32),
            pltpu.VMEM((_CHUNK, _D), jnp.float32),
            pltpu.SemaphoreType.DMA,
            pltpu.SemaphoreType.DMA,
            pltpu.SemaphoreType.DMA,
            pltpu.SemaphoreType.DMA,
            pltpu.SemaphoreType.DMA,
        ],
    )
    out = f(x, idx, tab)
    return out.reshape(_B, _L, _D)
